# Initial kernel scaffold; baseline (speedup 1.0000x reference)
import jax
import jax.numpy as jnp
from jax.experimental import pallas as pl


def kernel(x, Wg, bg, We_gate, be_gate, We_up, be_up, We_down, be_down, Wsu, bsu, Wsd, bsd):
    raise NotImplementedError("write your pallas kernel here")


# speedup vs baseline: 2.6934x; 2.6934x over previous
import jax
import jax.numpy as jnp
from jax.experimental import pallas as pl
from jax.experimental.pallas import tpu as pltpu

_TM = 256


def _silu(v):
    return v * jax.nn.sigmoid(v)


def _moe_body(x_ref, Wg_ref, bg_ref, Weg_ref, beg_ref, Weu_ref, beu_ref,
              Wed_ref, bed_ref, Wsu_ref, bsu_ref, Wsd_ref, bsd_ref, out_ref):
    x = x_ref[...]
    xb = x.astype(jnp.bfloat16)

    gate = jnp.dot(x, Wg_ref[...], preferred_element_type=jnp.float32)
    gate = gate + bg_ref[...]
    idx = jax.lax.broadcasted_iota(jnp.int32, gate.shape, 1)
    m1 = jnp.min(gate, axis=-1, keepdims=True)
    e1 = jnp.max(jnp.where(gate == m1, idx, -1), axis=-1, keepdims=True)
    g2 = jnp.where(idx == e1, jnp.inf, gate)
    m2 = jnp.min(g2, axis=-1, keepdims=True)
    e2 = jnp.max(jnp.where(g2 == m2, idx, -1), axis=-1, keepdims=True)
    excluded = (idx == e1) | (idx == e2)
    mx = jnp.max(gate, axis=-1, keepdims=True)
    ex = jnp.where(excluded, 0.0, jnp.exp(gate - mx))
    p = ex / jnp.sum(ex, axis=-1, keepdims=True)

    u = jnp.dot(xb, Wsu_ref[...], preferred_element_type=jnp.float32)
    a = _silu(u + bsu_ref[...]).astype(jnp.bfloat16)
    acc = jnp.dot(a, Wsd_ref[...], preferred_element_type=jnp.float32)
    acc = acc + bsd_ref[...]

    n_routed = Weg_ref.shape[0]
    for i in range(n_routed):
        g = jnp.dot(xb, Weg_ref[i], preferred_element_type=jnp.float32)
        g = g + beg_ref[i:i + 1]
        uu = jnp.dot(xb, Weu_ref[i], preferred_element_type=jnp.float32)
        uu = uu + beu_ref[i:i + 1]
        h = (_silu(g) * uu).astype(jnp.bfloat16)
        o = jnp.dot(h, Wed_ref[i], preferred_element_type=jnp.float32)
        o = o + bed_ref[i:i + 1]
        acc = acc + o * p[:, i:i + 1]

    out_ref[...] = acc


def _whole(shape):
    nd = len(shape)
    return pl.BlockSpec(shape, lambda i: (0,) * nd)


@jax.jit
def kernel(x, Wg, bg, We_gate, be_gate, We_up, be_up, We_down, be_down,
           Wsu, bsu, Wsd, bsd):
    B, S, H = x.shape
    T = B * S
    E = Wg.shape[1]
    xf = x.reshape(T, H)

    bf = jnp.bfloat16
    Wegb = We_gate.astype(bf)
    Weub = We_up.astype(bf)
    Wedb = We_down.astype(bf)
    Wsub = Wsu.astype(bf)
    Wsdb = Wsd.astype(bf)
    bg2 = bg.reshape(1, E)
    bsu2 = bsu.reshape(1, -1)
    bsd2 = bsd.reshape(1, -1)

    out = pl.pallas_call(
        _moe_body,
        grid=(T // _TM,),
        in_specs=[
            pl.BlockSpec((_TM, H), lambda i: (i, 0)),
            _whole(Wg.shape),
            _whole(bg2.shape),
            _whole(Wegb.shape),
            _whole(be_gate.shape),
            _whole(Weub.shape),
            _whole(be_up.shape),
            _whole(Wedb.shape),
            _whole(be_down.shape),
            _whole(Wsub.shape),
            _whole(bsu2.shape),
            _whole(Wsdb.shape),
            _whole(bsd2.shape),
        ],
        out_specs=pl.BlockSpec((_TM, H), lambda i: (i, 0)),
        out_shape=jax.ShapeDtypeStruct((T, H), jnp.float32),
    )(xf, Wg, bg2, Wegb, be_gate, Weub, be_up, Wedb, be_down,
      Wsub, bsu2, Wsdb, bsd2)
    return out.reshape(B, S, H)


# speedup vs baseline: 2.8901x; 1.0731x over previous
import jax
import jax.numpy as jnp
from jax.experimental import pallas as pl
from jax.experimental.pallas import tpu as pltpu

_TM = 256


def _silu(v):
    return v * jax.nn.sigmoid(v)


def _moe_body(x_ref, Wg_ref, bg_ref, Wup_ref, bup_ref, Wdn_ref, bed_ref,
              bsd_ref, out_ref, *, n_routed, ex):
    x = x_ref[...]
    xb = x.astype(jnp.bfloat16)
    nex = n_routed * ex

    gate = jnp.dot(x, Wg_ref[...], preferred_element_type=jnp.float32)
    gate = gate + bg_ref[...]
    idx = jax.lax.broadcasted_iota(jnp.int32, gate.shape, 1)
    m1 = jnp.min(gate, axis=-1, keepdims=True)
    e1 = jnp.max(jnp.where(gate == m1, idx, -1), axis=-1, keepdims=True)
    g2 = jnp.where(idx == e1, jnp.inf, gate)
    m2 = jnp.min(g2, axis=-1, keepdims=True)
    e2 = jnp.max(jnp.where(g2 == m2, idx, -1), axis=-1, keepdims=True)
    excluded = (idx == e1) | (idx == e2)
    mx = jnp.max(gate, axis=-1, keepdims=True)
    exv = jnp.where(excluded, 0.0, jnp.exp(gate - mx))
    p = exv / jnp.sum(exv, axis=-1, keepdims=True)

    R = jnp.dot(xb, Wup_ref[...], preferred_element_type=jnp.float32)
    R = R + bup_ref[...]

    blocks = []
    for i in range(n_routed):
        g = R[:, i * ex:(i + 1) * ex]
        u = R[:, nex + i * ex:nex + (i + 1) * ex]
        blocks.append((_silu(g) * u * p[:, i:i + 1]).astype(jnp.bfloat16))
    blocks.append(_silu(R[:, 2 * nex:]).astype(jnp.bfloat16))
    H2 = jnp.concatenate(blocks, axis=1)

    acc = jnp.dot(H2, Wdn_ref[...], preferred_element_type=jnp.float32)
    acc = acc + bsd_ref[...]
    acc = acc + jnp.dot(p[:, :n_routed], bed_ref[...],
                        preferred_element_type=jnp.float32)
    out_ref[...] = acc


def _whole(shape):
    nd = len(shape)
    return pl.BlockSpec(shape, lambda i: (0,) * nd)


@jax.jit
def kernel(x, Wg, bg, We_gate, be_gate, We_up, be_up, We_down, be_down,
           Wsu, bsu, Wsd, bsd):
    B, S, H = x.shape
    T = B * S
    E = Wg.shape[1]
    n_routed, _, ex = We_gate.shape
    nex = n_routed * ex
    nse = Wsu.shape[1]
    xf = x.reshape(T, H)

    bf = jnp.bfloat16
    Wup = jnp.concatenate(
        [We_gate.transpose(1, 0, 2).reshape(H, nex),
         We_up.transpose(1, 0, 2).reshape(H, nex),
         Wsu], axis=1).astype(bf)
    bup = jnp.concatenate(
        [be_gate.reshape(1, nex), be_up.reshape(1, nex),
         bsu.reshape(1, nse)], axis=1)
    Wdn = jnp.concatenate([We_down.reshape(nex, H), Wsd], axis=0).astype(bf)
    bg2 = bg.reshape(1, E)
    bsd2 = bsd.reshape(1, H)

    import functools
    body = functools.partial(_moe_body, n_routed=n_routed, ex=ex)

    out = pl.pallas_call(
        body,
        grid=(T // _TM,),
        in_specs=[
            pl.BlockSpec((_TM, H), lambda i: (i, 0)),
            _whole(Wg.shape),
            _whole(bg2.shape),
            _whole(Wup.shape),
            _whole(bup.shape),
            _whole(Wdn.shape),
            _whole(be_down.shape),
            _whole(bsd2.shape),
        ],
        out_specs=pl.BlockSpec((_TM, H), lambda i: (i, 0)),
        out_shape=jax.ShapeDtypeStruct((T, H), jnp.float32),
    )(xf, Wg, bg2, Wup, bup, Wdn, be_down, bsd2)
    return out.reshape(B, S, H)
